# baseline (speedup 1.0000x reference)
import functools

import jax
import jax.numpy as jnp
from jax import lax
from jax.experimental import pallas as pl
from jax.experimental.pallas import tpu as pltpu
from jax.experimental.pallas import tpu_sc as plsc

_NW = 32
_CH = 80


def _prep_body(r_ref, qc_ref, fq_ref, wm2t_ref, wma2t_ref, bma_ref,
               wxa2t_ref, bxa_ref, gam_ref, bet_ref,
               rbn_ref, rw_ref, ca_ref, c0_ref, c1_ref):
    r = r_ref[...]
    mu = jnp.mean(r, axis=0, keepdims=True)
    var = jnp.mean((r - mu) ** 2, axis=0, keepdims=True)
    rbn = (r - mu) * lax.rsqrt(var + 1e-5) * gam_ref[...] + bet_ref[...]
    rbn_ref[...] = rbn
    rw_ref[...] = jnp.dot(rbn, wm2t_ref[...], preferred_element_type=jnp.float32)
    ca_ref[...] = (jnp.dot(qc_ref[...], wma2t_ref[...],
                           preferred_element_type=jnp.float32) + bma_ref[...])
    cc = (jnp.dot(fq_ref[...], wxa2t_ref[...],
                  preferred_element_type=jnp.float32) + bxa_ref[...])
    c0_ref[...] = cc[0:1]
    c1_ref[...] = cc[1:2]


def _edge_body(he_ref, rwg_ref, wm1t_ref, bm_ref, wa1t_ref, ca_ref, wma_ref,
               wmess_ref, wraw_ref):
    he = he_ref[...]
    m = jnp.dot(he, wm1t_ref[...], preferred_element_type=jnp.float32)
    mess = jnp.tanh(m + rwg_ref[...] + bm_ref[...])
    v = jnp.dot(mess, wa1t_ref[...], preferred_element_type=jnp.float32) + ca_ref[...]
    v = jnp.where(v >= 0, v, 0.01 * v)
    coeff = jnp.sum(v * wma_ref[...], axis=1, keepdims=True)
    w = jnp.exp(coeff)
    wmess_ref[...] = mess * w
    coeff_row = lax.dot_general(wma_ref[...], v, (((1,), (1,)), ((), ())),
                                preferred_element_type=jnp.float32)
    wraw_ref[...] = jnp.exp(coeff_row)[None]


def _node_body(x_ref, acc_ref, ssum_ref, wx1t_ref, c0_ref, c1_ref, wxa_ref,
               out_ref):
    xb = x_ref[...]
    a = acc_ref[0] + acc_ref[1]
    s = ssum_ref[0] + ssum_ref[1]
    sm = a * jnp.where(s > 0, 1.0 / s, 0.0)
    wxa = wxa_ref[...]
    u0 = jnp.dot(xb, wx1t_ref[...], preferred_element_type=jnp.float32) + c0_ref[...]
    u0 = jnp.where(u0 >= 0, u0, 0.01 * u0)
    k0 = jnp.sum(u0 * wxa, axis=1, keepdims=True)
    u1 = jnp.dot(sm, wx1t_ref[...], preferred_element_type=jnp.float32) + c1_ref[...]
    u1 = jnp.where(u1 >= 0, u1, 0.01 * u1)
    k1 = jnp.sum(u1 * wxa, axis=1, keepdims=True)
    mx = jnp.maximum(k0, k1)
    e0 = jnp.exp(k0 - mx)
    e1 = jnp.exp(k1 - mx)
    w0 = e0 / (e0 + e1)
    out_ref[...] = w0 * xb + (1.0 - w0) * sm


def _make_gather(E, n, H):
    per_w = E // _NW
    nit = per_w // _CH
    npairs = (nit + 1) // 2
    idx_len = 2 * npairs * _CH + 2 * _CH
    pad = idx_len - per_w
    mesh = plsc.VectorSubcoreMesh(core_axis_name="c", subcore_axis_name="s")

    @functools.partial(
        pl.kernel, mesh=mesh,
        out_type=(jax.ShapeDtypeStruct((E + pad, H), jnp.float32),
                  jax.ShapeDtypeStruct((E + pad, H), jnp.float32)),
        scratch_types=[
            pltpu.VMEM((idx_len,), jnp.int32),
            pltpu.VMEM((idx_len,), jnp.int32),
            pltpu.VMEM((_CH, H), jnp.float32),
            pltpu.VMEM((_CH, H), jnp.float32),
            pltpu.VMEM((_CH, H), jnp.float32),
            pltpu.VMEM((_CH, H), jnp.float32),
            pltpu.SemaphoreType.DMA,
            pltpu.SemaphoreType.DMA,
            pltpu.SemaphoreType.DMA,
            pltpu.SemaphoreType.DMA,
            pltpu.SemaphoreType.DMA,
            pltpu.SemaphoreType.DMA,
        ],
    )
    def gather(x_hbm, rw_hbm, head_hbm, attr_hbm, he_out, rwg_out,
               idxh, idxr, bh0, bh1, br0, br1, sh0, sh1, sr0, sr1, sw0, sw1):
        wid = lax.axis_index("s") * 2 + lax.axis_index("c")
        base0 = pl.multiple_of(wid * per_w, 8)
        pltpu.sync_copy(head_hbm.at[pl.ds(base0, idx_len)], idxh)
        pltpu.sync_copy(attr_hbm.at[pl.ds(base0, idx_len)], idxr)

        def issue(j, bh, br, sh, sr):
            pltpu.async_copy(x_hbm.at[idxh.at[pl.ds(j * _CH, _CH)]], bh, sh)
            pltpu.async_copy(rw_hbm.at[idxr.at[pl.ds(j * _CH, _CH)]], br, sr)

        def wait(bh, br, sh, sr):
            pltpu.make_async_copy(x_hbm.at[pl.ds(0, _CH)], bh, sh).wait()
            pltpu.make_async_copy(rw_hbm.at[pl.ds(0, _CH)], br, sr).wait()

        def wb_start(base, bh, br, sw):
            pltpu.async_copy(bh, he_out.at[pl.ds(base, _CH)], sw)
            pltpu.async_copy(br, rwg_out.at[pl.ds(base, _CH)], sw)

        def wb_wait(bh, br, sw):
            pltpu.make_async_copy(bh, he_out.at[pl.ds(0, _CH)], sw).wait()
            pltpu.make_async_copy(br, rwg_out.at[pl.ds(0, _CH)], sw).wait()

        issue(0, bh0, br0, sh0, sr0)
        issue(1, bh1, br1, sh1, sr1)

        def body(g, carry):
            j0 = 2 * g
            base_a = pl.multiple_of(base0 + j0 * _CH, 8)
            base_b = pl.multiple_of(base_a + _CH, 8)
            wait(bh0, br0, sh0, sr0)
            wb_start(base_a, bh0, br0, sw0)
            wait(bh1, br1, sh1, sr1)
            wb_start(base_b, bh1, br1, sw1)
            wb_wait(bh0, br0, sw0)
            issue(j0 + 2, bh0, br0, sh0, sr0)
            wb_wait(bh1, br1, sw1)
            issue(j0 + 3, bh1, br1, sh1, sr1)
            return carry

        lax.fori_loop(0, npairs, body, 0)
        wait(bh0, br0, sh0, sr0)
        wait(bh1, br1, sh1, sr1)

    return gather, pad


def kernel(x, r, que_context, fin_que, edge_index, edge_attr,
           W_mess, b_mess, W_ma, b_ma, w_ma, W_xa, b_xa, w_xa,
           bn_gamma, bn_beta):
    n, H = x.shape
    E = edge_index.shape[1]
    R = r.shape[0]
    assert E % (_NW * _CH) == 0 and H == 128

    head = edge_index[0].astype(jnp.int32)
    tail = edge_index[1].astype(jnp.int32)
    attr = edge_attr.astype(jnp.int32)

    rbn, rW, ca, c0, c1 = pl.pallas_call(
        _prep_body,
        out_shape=(
            jax.ShapeDtypeStruct((R, H), jnp.float32),
            jax.ShapeDtypeStruct((R, H), jnp.float32),
            jax.ShapeDtypeStruct((1, H), jnp.float32),
            jax.ShapeDtypeStruct((1, H), jnp.float32),
            jax.ShapeDtypeStruct((1, H), jnp.float32),
        ),
    )(r, que_context, fin_que[0],
      W_mess[:, H:].T, W_ma[:, H:].T, b_ma.reshape(1, H),
      W_xa[:, H:].T, b_xa.reshape(1, H),
      bn_gamma.reshape(1, H), bn_beta.reshape(1, H))

    gfn, gpad = _make_gather(E, n, H)
    head_p = jnp.concatenate([head, jnp.zeros((gpad,), jnp.int32)])
    attr_p = jnp.concatenate([attr, jnp.zeros((gpad,), jnp.int32)])
    he, rwg = gfn(x, rW, head_p, attr_p)

    B = 2560
    grid = (E // B,)
    wmess, wraw = pl.pallas_call(
        _edge_body,
        grid=grid,
        in_specs=[
            pl.BlockSpec((B, H), lambda i: (i, 0)),
            pl.BlockSpec((B, H), lambda i: (i, 0)),
            pl.BlockSpec((H, H), lambda i: (0, 0)),
            pl.BlockSpec((1, H), lambda i: (0, 0)),
            pl.BlockSpec((H, H), lambda i: (0, 0)),
            pl.BlockSpec((1, H), lambda i: (0, 0)),
            pl.BlockSpec((1, H), lambda i: (0, 0)),
        ],
        out_specs=(
            pl.BlockSpec((B, H), lambda i: (i, 0)),
            pl.BlockSpec((1, 1, B), lambda i: (i, 0, 0)),
        ),
        out_shape=(
            jax.ShapeDtypeStruct((E + _CH, H), jnp.float32),
            jax.ShapeDtypeStruct((E // B, 1, B), jnp.float32),
        ),
    )(he, rwg, W_mess[:, :H].T, b_mess.reshape(1, H),
      W_ma[:, :H].T, ca, w_ma)

    np_ = 10240
    zacc = jnp.zeros((40, H), jnp.float32)
    zs1 = jnp.zeros((40,), jnp.float32)
    wraw1 = jnp.concatenate([wraw.reshape(E), jnp.zeros((_CH,), jnp.float32)])
    tail_p = jnp.concatenate([tail, jnp.zeros((_CH,), jnp.int32)])

    mesh = plsc.VectorSubcoreMesh(core_axis_name="c", subcore_axis_name="s")
    per_w = E // _NW
    nit = per_w // _CH
    npr = (nit - 1) // 2
    half = E // 2

    @functools.partial(
        pl.kernel, mesh=mesh,
        out_type=(jax.ShapeDtypeStruct((2 * np_, H), jnp.float32),
                  jax.ShapeDtypeStruct((2 * np_,), jnp.float32)),
        scratch_types=[
            pltpu.VMEM((2, _CH), jnp.int32),
            pltpu.VMEM((_CH, H), jnp.float32),
            pltpu.VMEM((_CH, H), jnp.float32),
            pltpu.VMEM((_CH,), jnp.float32),
            pltpu.VMEM((_CH,), jnp.float32),
            pltpu.VMEM((40, H), jnp.float32),
            pltpu.VMEM((40,), jnp.float32),
            pltpu.VMEM_SHARED((np_, H), jnp.float32),
            pltpu.VMEM_SHARED((np_,), jnp.float32),
            pltpu.SemaphoreType.DMA,
            pltpu.SemaphoreType.DMA,
        ],
    )
    def _sc_scatter(wmess_hbm, wraw_hbm, tail_hbm, z_hbm, zs_hbm,
                    acc_out, ssum_out,
                    idx2, rb0, rb1, wb0, wb1, zb_v, zs_v,
                    acc_sh, ssum_sh, ld0, ld1):
        c = lax.axis_index("c")
        s = lax.axis_index("s")
        pltpu.sync_copy(z_hbm, zb_v)
        pltpu.sync_copy(zs_hbm, zs_v)

        def zbody(j, carry):
            r0 = pl.multiple_of(s * 640 + j * 40, 8)
            pltpu.sync_copy(zb_v, acc_sh.at[pl.ds(r0, 40)])
            pltpu.sync_copy(zs_v, ssum_sh.at[pl.ds(r0, 40)])
            return carry

        lax.fori_loop(0, 16, zbody, 0)
        plsc.subcore_barrier()
        base0 = pl.multiple_of(c * half + s * per_w, 8)

        def issue(j, b, rb, wb, sem):
            base = pl.multiple_of(base0 + j * _CH, 8)
            pltpu.async_copy(tail_hbm.at[pl.ds(base, _CH)], idx2.at[b], sem)
            pltpu.async_copy(wmess_hbm.at[pl.ds(base, _CH)], rb, sem)
            pltpu.async_copy(wraw_hbm.at[pl.ds(base, _CH)], wb, sem)

        def wait(b, rb, wb, sem):
            pltpu.make_async_copy(tail_hbm.at[pl.ds(0, _CH)], idx2.at[b], sem).wait()
            pltpu.make_async_copy(wmess_hbm.at[pl.ds(0, _CH)], rb, sem).wait()
            pltpu.make_async_copy(wraw_hbm.at[pl.ds(0, _CH)], wb, sem).wait()

        def scat(b, rb, wb):
            pltpu.sync_copy(rb, acc_sh.at[idx2.at[b]], add=True)
            pltpu.sync_copy(wb, ssum_sh.at[idx2.at[b]], add=True)

        issue(0, 0, rb0, wb0, ld0)
        issue(1, 1, rb1, wb1, ld1)

        def body(g, carry):
            j0 = 2 * g
            wait(0, rb0, wb0, ld0)
            scat(0, rb0, wb0)
            issue(j0 + 2, 0, rb0, wb0, ld0)
            wait(1, rb1, wb1, ld1)
            scat(1, rb1, wb1)
            issue(j0 + 3, 1, rb1, wb1, ld1)
            return carry

        lax.fori_loop(0, npr, body, 0)
        wait(0, rb0, wb0, ld0)
        scat(0, rb0, wb0)
        wait(1, rb1, wb1, ld1)
        plsc.subcore_barrier()

        def fbody(j, carry):
            r0 = pl.multiple_of(s * 640 + j * 40, 8)
            o0 = pl.multiple_of(c * np_ + s * 640 + j * 40, 8)
            pltpu.sync_copy(acc_sh.at[pl.ds(r0, 40)], zb_v)
            pltpu.sync_copy(zb_v, acc_out.at[pl.ds(o0, 40)])
            pltpu.sync_copy(ssum_sh.at[pl.ds(r0, 40)], zs_v)
            pltpu.sync_copy(zs_v, ssum_out.at[pl.ds(o0, 40)])
            return carry

        lax.fori_loop(0, 16, fbody, 0)

    acc_flat, ssum_flat = _sc_scatter(wmess, wraw1, tail_p, zacc, zs1)
    acc = acc_flat.reshape(2, np_, H)[:, :n]
    ssum = ssum_flat.reshape(2, np_, 1)[:, :n]

    Bn = 2000
    x_out = pl.pallas_call(
        _node_body,
        grid=(n // Bn,),
        in_specs=[
            pl.BlockSpec((Bn, H), lambda i: (i, 0)),
            pl.BlockSpec((2, Bn, H), lambda i: (0, i, 0)),
            pl.BlockSpec((2, Bn, 1), lambda i: (0, i, 0)),
            pl.BlockSpec((H, H), lambda i: (0, 0)),
            pl.BlockSpec((1, H), lambda i: (0, 0)),
            pl.BlockSpec((1, H), lambda i: (0, 0)),
            pl.BlockSpec((1, H), lambda i: (0, 0)),
        ],
        out_specs=pl.BlockSpec((Bn, H), lambda i: (i, 0)),
        out_shape=jax.ShapeDtypeStruct((n, H), jnp.float32),
    )(x, acc, ssum, W_xa[:, :H].T, c0, c1, w_xa)

    return (x_out, rbn)


# speedup vs baseline: 1.2245x; 1.2245x over previous
import functools

import jax
import jax.numpy as jnp
from jax import lax
from jax.experimental import pallas as pl
from jax.experimental.pallas import tpu as pltpu
from jax.experimental.pallas import tpu_sc as plsc

_NW = 32
_CH = 80


def _prep_body(r_ref, qc_ref, fq_ref, wm2t_ref, wma2t_ref, bma_ref,
               wxa2t_ref, bxa_ref, gam_ref, bet_ref,
               rbn_ref, rw_ref, ca_ref, c0_ref, c1_ref):
    r = r_ref[...]
    mu = jnp.mean(r, axis=0, keepdims=True)
    var = jnp.mean((r - mu) ** 2, axis=0, keepdims=True)
    rbn = (r - mu) * lax.rsqrt(var + 1e-5) * gam_ref[...] + bet_ref[...]
    rbn_ref[...] = rbn
    rw_ref[...] = jnp.dot(rbn, wm2t_ref[...], preferred_element_type=jnp.float32)
    ca_ref[...] = (jnp.dot(qc_ref[...], wma2t_ref[...],
                           preferred_element_type=jnp.float32) + bma_ref[...])
    cc = (jnp.dot(fq_ref[...], wxa2t_ref[...],
                  preferred_element_type=jnp.float32) + bxa_ref[...])
    c0_ref[...] = cc[0:1]
    c1_ref[...] = cc[1:2]


def _edge_body(he_ref, rwg_ref, wm1t_ref, bm_ref, wa1t_ref, ca_ref, wma_ref,
               wmess_ref, wraw_ref):
    he = he_ref[...]
    m = jnp.dot(he, wm1t_ref[...], preferred_element_type=jnp.float32)
    mess = jnp.tanh(m + rwg_ref[...] + bm_ref[...])
    v = jnp.dot(mess, wa1t_ref[...], preferred_element_type=jnp.float32) + ca_ref[...]
    v = jnp.where(v >= 0, v, 0.01 * v)
    coeff = jnp.sum(v * wma_ref[...], axis=1, keepdims=True)
    w = jnp.exp(coeff)
    wmess_ref[...] = mess * w
    coeff_row = lax.dot_general(wma_ref[...], v, (((1,), (1,)), ((), ())),
                                preferred_element_type=jnp.float32)
    wraw_ref[...] = jnp.exp(coeff_row)[None]


def _node_body(x_ref, acc_ref, ssum_ref, wx1t_ref, c0_ref, c1_ref, wxa_ref,
               out_ref):
    xb = x_ref[...]
    a = acc_ref[0] + acc_ref[1]
    s = ssum_ref[0] + ssum_ref[1]
    sm = a * jnp.where(s > 0, 1.0 / s, 0.0)
    wxa = wxa_ref[...]
    u0 = jnp.dot(xb, wx1t_ref[...], preferred_element_type=jnp.float32) + c0_ref[...]
    u0 = jnp.where(u0 >= 0, u0, 0.01 * u0)
    k0 = jnp.sum(u0 * wxa, axis=1, keepdims=True)
    u1 = jnp.dot(sm, wx1t_ref[...], preferred_element_type=jnp.float32) + c1_ref[...]
    u1 = jnp.where(u1 >= 0, u1, 0.01 * u1)
    k1 = jnp.sum(u1 * wxa, axis=1, keepdims=True)
    mx = jnp.maximum(k0, k1)
    e0 = jnp.exp(k0 - mx)
    e1 = jnp.exp(k1 - mx)
    w0 = e0 / (e0 + e1)
    out_ref[...] = w0 * xb + (1.0 - w0) * sm


def _make_gather(E, n, H):
    per_w = E // _NW
    nit = per_w // _CH
    npairs = (nit + 1) // 2
    idx_len = 2 * npairs * _CH + 2 * _CH
    pad = idx_len - per_w
    mesh = plsc.VectorSubcoreMesh(core_axis_name="c", subcore_axis_name="s")

    @functools.partial(
        pl.kernel, mesh=mesh,
        out_type=(jax.ShapeDtypeStruct((E + pad, H), jnp.float32),
                  jax.ShapeDtypeStruct((E + pad, H), jnp.float32)),
        scratch_types=[
            pltpu.VMEM((idx_len,), jnp.int32),
            pltpu.VMEM((idx_len,), jnp.int32),
            pltpu.VMEM((_CH, H), jnp.float32),
            pltpu.VMEM((_CH, H), jnp.float32),
            pltpu.VMEM((_CH, H), jnp.float32),
            pltpu.VMEM((_CH, H), jnp.float32),
            pltpu.VMEM((32, H), jnp.float32),
            pltpu.VMEM_SHARED((512, H), jnp.float32),
            pltpu.SemaphoreType.DMA,
            pltpu.SemaphoreType.DMA,
            pltpu.SemaphoreType.DMA,
            pltpu.SemaphoreType.DMA,
            pltpu.SemaphoreType.DMA,
            pltpu.SemaphoreType.DMA,
        ],
    )
    def gather(x_hbm, rw_hbm, head_hbm, attr_hbm, he_out, rwg_out,
               idxh, idxr, bh0, bh1, br0, br1, rstage, rw_sh,
               sh0, sh1, sr0, sr1, sw0, sw1):
        wid = lax.axis_index("s") * 2 + lax.axis_index("c")
        s = lax.axis_index("s")
        base0 = pl.multiple_of(wid * per_w, 8)
        r0 = pl.multiple_of(s * 32, 8)
        pltpu.sync_copy(rw_hbm.at[pl.ds(r0, 32)], rstage)
        pltpu.sync_copy(rstage, rw_sh.at[pl.ds(r0, 32)])
        pltpu.sync_copy(head_hbm.at[pl.ds(base0, idx_len)], idxh)
        pltpu.sync_copy(attr_hbm.at[pl.ds(base0, idx_len)], idxr)
        plsc.subcore_barrier()

        def issue(j, bh, br, sh, sr):
            pltpu.async_copy(x_hbm.at[idxh.at[pl.ds(j * _CH, _CH)]], bh, sh)
            pltpu.async_copy(rw_sh.at[idxr.at[pl.ds(j * _CH, _CH)]], br, sr)

        def wait(bh, br, sh, sr):
            pltpu.make_async_copy(x_hbm.at[pl.ds(0, _CH)], bh, sh).wait()
            pltpu.make_async_copy(x_hbm.at[pl.ds(0, _CH)], br, sr).wait()

        def wb_start(base, bh, br, sw):
            pltpu.async_copy(bh, he_out.at[pl.ds(base, _CH)], sw)
            pltpu.async_copy(br, rwg_out.at[pl.ds(base, _CH)], sw)

        def wb_wait(bh, br, sw):
            pltpu.make_async_copy(bh, he_out.at[pl.ds(0, _CH)], sw).wait()
            pltpu.make_async_copy(br, rwg_out.at[pl.ds(0, _CH)], sw).wait()

        issue(0, bh0, br0, sh0, sr0)
        issue(1, bh1, br1, sh1, sr1)

        def body(g, carry):
            j0 = 2 * g
            base_a = pl.multiple_of(base0 + j0 * _CH, 8)
            base_b = pl.multiple_of(base_a + _CH, 8)
            wait(bh0, br0, sh0, sr0)
            wb_start(base_a, bh0, br0, sw0)
            wait(bh1, br1, sh1, sr1)
            wb_start(base_b, bh1, br1, sw1)
            wb_wait(bh0, br0, sw0)
            issue(j0 + 2, bh0, br0, sh0, sr0)
            wb_wait(bh1, br1, sw1)
            issue(j0 + 3, bh1, br1, sh1, sr1)
            return carry

        lax.fori_loop(0, npairs, body, 0)
        wait(bh0, br0, sh0, sr0)
        wait(bh1, br1, sh1, sr1)

    return gather, pad


def kernel(x, r, que_context, fin_que, edge_index, edge_attr,
           W_mess, b_mess, W_ma, b_ma, w_ma, W_xa, b_xa, w_xa,
           bn_gamma, bn_beta):
    n, H = x.shape
    E = edge_index.shape[1]
    R = r.shape[0]
    assert E % (_NW * _CH) == 0 and H == 128

    head = edge_index[0].astype(jnp.int32)
    tail = edge_index[1].astype(jnp.int32)
    attr = edge_attr.astype(jnp.int32)

    rbn, rW, ca, c0, c1 = pl.pallas_call(
        _prep_body,
        out_shape=(
            jax.ShapeDtypeStruct((R, H), jnp.float32),
            jax.ShapeDtypeStruct((R, H), jnp.float32),
            jax.ShapeDtypeStruct((1, H), jnp.float32),
            jax.ShapeDtypeStruct((1, H), jnp.float32),
            jax.ShapeDtypeStruct((1, H), jnp.float32),
        ),
    )(r, que_context, fin_que[0],
      W_mess[:, H:].T, W_ma[:, H:].T, b_ma.reshape(1, H),
      W_xa[:, H:].T, b_xa.reshape(1, H),
      bn_gamma.reshape(1, H), bn_beta.reshape(1, H))

    gfn, gpad = _make_gather(E, n, H)
    head_p = jnp.concatenate([head, jnp.zeros((gpad,), jnp.int32)])
    attr_p = jnp.concatenate([attr, jnp.zeros((gpad,), jnp.int32)])
    rW_p = jnp.concatenate([rW, jnp.zeros((512 - R, H), jnp.float32)])
    he, rwg = gfn(x, rW_p, head_p, attr_p)

    B = 2560
    grid = (E // B,)
    wmess, wraw = pl.pallas_call(
        _edge_body,
        grid=grid,
        in_specs=[
            pl.BlockSpec((B, H), lambda i: (i, 0)),
            pl.BlockSpec((B, H), lambda i: (i, 0)),
            pl.BlockSpec((H, H), lambda i: (0, 0)),
            pl.BlockSpec((1, H), lambda i: (0, 0)),
            pl.BlockSpec((H, H), lambda i: (0, 0)),
            pl.BlockSpec((1, H), lambda i: (0, 0)),
            pl.BlockSpec((1, H), lambda i: (0, 0)),
        ],
        out_specs=(
            pl.BlockSpec((B, H), lambda i: (i, 0)),
            pl.BlockSpec((1, 1, B), lambda i: (i, 0, 0)),
        ),
        out_shape=(
            jax.ShapeDtypeStruct((E + _CH, H), jnp.float32),
            jax.ShapeDtypeStruct((E // B, 1, B), jnp.float32),
        ),
    )(he, rwg, W_mess[:, :H].T, b_mess.reshape(1, H),
      W_ma[:, :H].T, ca, w_ma)

    np_ = 10240
    zacc = jnp.zeros((40, H), jnp.float32)
    zs1 = jnp.zeros((40,), jnp.float32)
    wraw1 = jnp.concatenate([wraw.reshape(E), jnp.zeros((_CH,), jnp.float32)])
    tail_p = jnp.concatenate([tail, jnp.zeros((_CH,), jnp.int32)])

    mesh = plsc.VectorSubcoreMesh(core_axis_name="c", subcore_axis_name="s")
    per_w = E // _NW
    nit = per_w // _CH
    npr = (nit - 1) // 2
    half = E // 2

    @functools.partial(
        pl.kernel, mesh=mesh,
        out_type=(jax.ShapeDtypeStruct((2 * np_, H), jnp.float32),
                  jax.ShapeDtypeStruct((2 * np_,), jnp.float32)),
        scratch_types=[
            pltpu.VMEM((2, _CH), jnp.int32),
            pltpu.VMEM((_CH, H), jnp.float32),
            pltpu.VMEM((_CH, H), jnp.float32),
            pltpu.VMEM((_CH,), jnp.float32),
            pltpu.VMEM((_CH,), jnp.float32),
            pltpu.VMEM((40, H), jnp.float32),
            pltpu.VMEM((40,), jnp.float32),
            pltpu.VMEM_SHARED((np_, H), jnp.float32),
            pltpu.VMEM_SHARED((np_,), jnp.float32),
            pltpu.SemaphoreType.DMA,
            pltpu.SemaphoreType.DMA,
        ],
    )
    def _sc_scatter(wmess_hbm, wraw_hbm, tail_hbm, z_hbm, zs_hbm,
                    acc_out, ssum_out,
                    idx2, rb0, rb1, wb0, wb1, zb_v, zs_v,
                    acc_sh, ssum_sh, ld0, ld1):
        c = lax.axis_index("c")
        s = lax.axis_index("s")
        pltpu.sync_copy(z_hbm, zb_v)
        pltpu.sync_copy(zs_hbm, zs_v)

        def zbody(j, carry):
            r0 = pl.multiple_of(s * 640 + j * 40, 8)
            pltpu.sync_copy(zb_v, acc_sh.at[pl.ds(r0, 40)])
            pltpu.sync_copy(zs_v, ssum_sh.at[pl.ds(r0, 40)])
            return carry

        lax.fori_loop(0, 16, zbody, 0)
        plsc.subcore_barrier()
        base0 = pl.multiple_of(c * half + s * per_w, 8)

        def issue(j, b, rb, wb, sem):
            base = pl.multiple_of(base0 + j * _CH, 8)
            pltpu.async_copy(tail_hbm.at[pl.ds(base, _CH)], idx2.at[b], sem)
            pltpu.async_copy(wmess_hbm.at[pl.ds(base, _CH)], rb, sem)
            pltpu.async_copy(wraw_hbm.at[pl.ds(base, _CH)], wb, sem)

        def wait(b, rb, wb, sem):
            pltpu.make_async_copy(tail_hbm.at[pl.ds(0, _CH)], idx2.at[b], sem).wait()
            pltpu.make_async_copy(wmess_hbm.at[pl.ds(0, _CH)], rb, sem).wait()
            pltpu.make_async_copy(wraw_hbm.at[pl.ds(0, _CH)], wb, sem).wait()

        def scat(b, rb, wb):
            pltpu.sync_copy(rb, acc_sh.at[idx2.at[b]], add=True)
            pltpu.sync_copy(wb, ssum_sh.at[idx2.at[b]], add=True)

        issue(0, 0, rb0, wb0, ld0)
        issue(1, 1, rb1, wb1, ld1)

        def body(g, carry):
            j0 = 2 * g
            wait(0, rb0, wb0, ld0)
            scat(0, rb0, wb0)
            issue(j0 + 2, 0, rb0, wb0, ld0)
            wait(1, rb1, wb1, ld1)
            scat(1, rb1, wb1)
            issue(j0 + 3, 1, rb1, wb1, ld1)
            return carry

        lax.fori_loop(0, npr, body, 0)
        wait(0, rb0, wb0, ld0)
        scat(0, rb0, wb0)
        wait(1, rb1, wb1, ld1)
        plsc.subcore_barrier()

        def fbody(j, carry):
            r0 = pl.multiple_of(s * 640 + j * 40, 8)
            o0 = pl.multiple_of(c * np_ + s * 640 + j * 40, 8)
            pltpu.sync_copy(acc_sh.at[pl.ds(r0, 40)], zb_v)
            pltpu.sync_copy(zb_v, acc_out.at[pl.ds(o0, 40)])
            pltpu.sync_copy(ssum_sh.at[pl.ds(r0, 40)], zs_v)
            pltpu.sync_copy(zs_v, ssum_out.at[pl.ds(o0, 40)])
            return carry

        lax.fori_loop(0, 16, fbody, 0)

    acc_flat, ssum_flat = _sc_scatter(wmess, wraw1, tail_p, zacc, zs1)
    acc = acc_flat.reshape(2, np_, H)[:, :n]
    ssum = ssum_flat.reshape(2, np_, 1)[:, :n]

    Bn = 2000
    x_out = pl.pallas_call(
        _node_body,
        grid=(n // Bn,),
        in_specs=[
            pl.BlockSpec((Bn, H), lambda i: (i, 0)),
            pl.BlockSpec((2, Bn, H), lambda i: (0, i, 0)),
            pl.BlockSpec((2, Bn, 1), lambda i: (0, i, 0)),
            pl.BlockSpec((H, H), lambda i: (0, 0)),
            pl.BlockSpec((1, H), lambda i: (0, 0)),
            pl.BlockSpec((1, H), lambda i: (0, 0)),
            pl.BlockSpec((1, H), lambda i: (0, 0)),
        ],
        out_specs=pl.BlockSpec((Bn, H), lambda i: (i, 0)),
        out_shape=jax.ShapeDtypeStruct((n, H), jnp.float32),
    )(x, acc, ssum, W_xa[:, :H].T, c0, c1, w_xa)

    return (x_out, rbn)
